# Initial kernel scaffold; baseline (speedup 1.0000x reference)
import jax
import jax.numpy as jnp
from jax.experimental import pallas as pl

HID = 2048
DFF = 768
NE = 16
NK = 8


def _router_body(x_ref, gw_ref, logits_ref, wdense_ref):
    x = x_ref[...]
    gw = gw_ref[...]
    logits = jax.lax.dot_general(
        x, gw, (((1,), (1,)), ((), ())),
        preferred_element_type=jnp.float32,
        precision=jax.lax.Precision.HIGHEST)
    logits_ref[...] = logits
    m = jnp.max(logits, axis=1, keepdims=True)
    ex = jnp.exp(logits - m)
    probs = ex / jnp.sum(ex, axis=1, keepdims=True)
    iota = jax.lax.broadcasted_iota(jnp.int32, probs.shape, 1)
    cur = probs
    wsel = jnp.zeros_like(probs)
    for _ in range(NK):
        mx = jnp.max(cur, axis=1, keepdims=True)
        cand = jnp.where(cur == mx, iota, NE)
        first = jnp.min(cand, axis=1, keepdims=True)
        onehot = iota == first
        wsel = jnp.where(onehot, probs, wsel)
        cur = jnp.where(onehot, -jnp.inf, cur)
    wdense_ref[...] = wsel / jnp.sum(wsel, axis=1, keepdims=True)


def _dense_ffn_body(xb_ref, wdense_ref, gwb_ref, uwb_ref, dwb_ref, out_ref):
    e = pl.program_id(1)
    xb = xb_ref[...]
    g = jax.lax.dot_general(xb, gwb_ref[0], (((1,), (1,)), ((), ())),
                            preferred_element_type=jnp.float32)
    u = jax.lax.dot_general(xb, uwb_ref[0], (((1,), (1,)), ((), ())),
                            preferred_element_type=jnp.float32)
    h = (g * jax.nn.sigmoid(g) * u).astype(jnp.bfloat16)
    y = jax.lax.dot_general(h, dwb_ref[0], (((1,), (1,)), ((), ())),
                            preferred_element_type=jnp.float32)
    w = wdense_ref[...]
    iota = jax.lax.broadcasted_iota(jnp.int32, w.shape, 1)
    wcol = jnp.sum(jnp.where(iota == e, w, 0.0), axis=1, keepdims=True)
    contrib = y * wcol

    @pl.when(e == 0)
    def _init():
        out_ref[...] = contrib

    @pl.when(e != 0)
    def _acc():
        out_ref[...] += contrib


def kernel(hidden_states, gate_w, gate_ws, up_ws, down_ws):
    bsz, seq, hd = hidden_states.shape
    T = bsz * seq
    x = hidden_states.reshape(T, hd)
    xb = x.astype(jnp.bfloat16)
    gwb = gate_ws.astype(jnp.bfloat16)
    uwb = up_ws.astype(jnp.bfloat16)
    dwb = down_ws.astype(jnp.bfloat16)

    TMR = 1024
    logits, wdense = pl.pallas_call(
        _router_body,
        grid=(T // TMR,),
        in_specs=[pl.BlockSpec((TMR, HID), lambda t: (t, 0)),
                  pl.BlockSpec((NE, HID), lambda t: (0, 0))],
        out_specs=[pl.BlockSpec((TMR, NE), lambda t: (t, 0)),
                   pl.BlockSpec((TMR, NE), lambda t: (t, 0))],
        out_shape=[jax.ShapeDtypeStruct((T, NE), jnp.float32),
                   jax.ShapeDtypeStruct((T, NE), jnp.float32)],
    )(x, gate_w)

    TM = 1024
    final = pl.pallas_call(
        _dense_ffn_body,
        grid=(T // TM, NE),
        in_specs=[pl.BlockSpec((TM, HID), lambda t, e: (t, 0)),
                  pl.BlockSpec((TM, NE), lambda t, e: (t, 0)),
                  pl.BlockSpec((1, DFF, HID), lambda t, e: (e, 0, 0)),
                  pl.BlockSpec((1, DFF, HID), lambda t, e: (e, 0, 0)),
                  pl.BlockSpec((1, HID, DFF), lambda t, e: (e, 0, 0))],
        out_specs=pl.BlockSpec((TM, HID), lambda t, e: (t, 0)),
        out_shape=jax.ShapeDtypeStruct((T, HID), jnp.float32),
    )(xb, wdense, gwb, uwb, dwb)

    return final.reshape(bsz, seq, hd), logits


# speedup vs baseline: 1.3343x; 1.3343x over previous
import jax
import jax.numpy as jnp
from jax.experimental import pallas as pl

HID = 2048
DFF = 768
NE = 16
NK = 8


def _router_body(x_ref, gw_ref, logits_ref, wdense_ref):
    x = x_ref[...]
    gw = gw_ref[...]
    logits = jax.lax.dot_general(
        x, gw, (((1,), (1,)), ((), ())),
        preferred_element_type=jnp.float32,
        precision=jax.lax.Precision.DEFAULT)
    logits_ref[...] = logits
    m = jnp.max(logits, axis=1, keepdims=True)
    ex = jnp.exp(logits - m)
    probs = ex / jnp.sum(ex, axis=1, keepdims=True)
    iota = jax.lax.broadcasted_iota(jnp.int32, probs.shape, 1)
    cur = probs
    wsel = jnp.zeros_like(probs)
    for _ in range(NK):
        mx = jnp.max(cur, axis=1, keepdims=True)
        cand = jnp.where(cur == mx, iota, NE)
        first = jnp.min(cand, axis=1, keepdims=True)
        onehot = iota == first
        wsel = jnp.where(onehot, probs, wsel)
        cur = jnp.where(onehot, -jnp.inf, cur)
    wdense_ref[...] = wsel / jnp.sum(wsel, axis=1, keepdims=True)


def _dense_ffn_body(xb_ref, wdense_ref, gwb_ref, uwb_ref, dwb_ref, out_ref):
    e = pl.program_id(1)
    xb = xb_ref[...]
    g = jax.lax.dot_general(xb, gwb_ref[0], (((1,), (1,)), ((), ())),
                            preferred_element_type=jnp.float32)
    u = jax.lax.dot_general(xb, uwb_ref[0], (((1,), (1,)), ((), ())),
                            preferred_element_type=jnp.float32)
    h = (g * jax.nn.sigmoid(g) * u).astype(jnp.bfloat16)
    y = jax.lax.dot_general(h, dwb_ref[0], (((1,), (1,)), ((), ())),
                            preferred_element_type=jnp.float32)
    w = wdense_ref[...]
    iota = jax.lax.broadcasted_iota(jnp.int32, w.shape, 1)
    wcol = jnp.sum(jnp.where(iota == e, w, 0.0), axis=1, keepdims=True)
    contrib = y * wcol

    @pl.when(e == 0)
    def _init():
        out_ref[...] = contrib

    @pl.when(e != 0)
    def _acc():
        out_ref[...] += contrib


def kernel(hidden_states, gate_w, gate_ws, up_ws, down_ws):
    bsz, seq, hd = hidden_states.shape
    T = bsz * seq
    x = hidden_states.reshape(T, hd)
    xb = x.astype(jnp.bfloat16)
    gwb = gate_ws.astype(jnp.bfloat16)
    uwb = up_ws.astype(jnp.bfloat16)
    dwb = down_ws.astype(jnp.bfloat16)

    TMR = 1024
    logits, wdense = pl.pallas_call(
        _router_body,
        grid=(T // TMR,),
        in_specs=[pl.BlockSpec((TMR, HID), lambda t: (t, 0)),
                  pl.BlockSpec((NE, HID), lambda t: (0, 0))],
        out_specs=[pl.BlockSpec((TMR, NE), lambda t: (t, 0)),
                   pl.BlockSpec((TMR, NE), lambda t: (t, 0))],
        out_shape=[jax.ShapeDtypeStruct((T, NE), jnp.float32),
                   jax.ShapeDtypeStruct((T, NE), jnp.float32)],
    )(x, gate_w)

    TM = 1024
    final = pl.pallas_call(
        _dense_ffn_body,
        grid=(T // TM, NE),
        in_specs=[pl.BlockSpec((TM, HID), lambda t, e: (t, 0)),
                  pl.BlockSpec((TM, NE), lambda t, e: (t, 0)),
                  pl.BlockSpec((1, DFF, HID), lambda t, e: (e, 0, 0)),
                  pl.BlockSpec((1, DFF, HID), lambda t, e: (e, 0, 0)),
                  pl.BlockSpec((1, HID, DFF), lambda t, e: (e, 0, 0))],
        out_specs=pl.BlockSpec((TM, HID), lambda t, e: (t, 0)),
        out_shape=jax.ShapeDtypeStruct((T, HID), jnp.float32),
    )(xb, wdense, gwb, uwb, dwb)

    return final.reshape(bsz, seq, hd), logits
